# baseline (speedup 1.0000x reference)
import functools

import jax
import jax.numpy as jnp
from jax.experimental import pallas as pl


def _pass1_body(xyz1_ref, bmat_ref, iota_ref, p2_ref, p1_ref, w0t_ref, b0_ref,
                t0_ref, stats_ref):
    nb = xyz1_ref.shape[1]
    m = bmat_ref.shape[2]
    c2 = p2_ref.shape[2]

    xyz1b = xyz1_ref[0]
    x1sq = jnp.sum(xyz1b * xyz1b, axis=1, keepdims=True)
    amat = jnp.concatenate(
        [jnp.ones((nb, 1), jnp.float32), x1sq, -2.0 * xyz1b,
         jnp.zeros((nb, 3), jnp.float32)], axis=1)
    d2 = jnp.dot(amat, bmat_ref[0],
                 preferred_element_type=jnp.float32)

    key = jax.lax.bitcast_convert_type(
        (jax.lax.bitcast_convert_type(d2, jnp.int32)
         & jnp.int32(-2048)) | iota_ref[...], jnp.float32)
    inf = jnp.float32(jnp.inf)

    k1 = jnp.min(key, axis=1, keepdims=True)
    eq1 = key == k1
    key2 = jnp.where(eq1, inf, key)
    k2 = jnp.min(key2, axis=1, keepdims=True)
    eq2 = key2 == k2
    key3 = jnp.where(eq2, inf, key2)
    k3 = jnp.min(key3, axis=1, keepdims=True)
    eq3 = key3 == k3

    def weight(k):
        v = jax.lax.bitcast_convert_type(
            jax.lax.bitcast_convert_type(k, jnp.int32) & jnp.int32(-2048),
            jnp.float32)
        return 1.0 / jnp.clip(v, 0.0, 1e-10)

    r1, r2, r3 = weight(k1), weight(k2), weight(k3)
    inv_norm = 1.0 / (r1 + r2 + r3)
    w1, w2, w3 = r1 * inv_norm, r2 * inv_norm, r3 * inv_norm
    zero = jnp.zeros((nb, m), jnp.float32)
    wmat = jnp.where(eq1, w1,
                     jnp.where(eq2, w2, jnp.where(eq3, w3, zero)))
    interp = jnp.dot(wmat, p2_ref[0],
                     preferred_element_type=jnp.float32)
    t0 = (jnp.dot(interp, w0t_ref[:c2, :],
                  preferred_element_type=jnp.float32)
          + jnp.dot(p1_ref[0], w0t_ref[c2:, :],
                    preferred_element_type=jnp.float32)
          + b0_ref[0:1, :])
    t0_ref[...] = t0

    first = (pl.program_id(0) == 0) & (pl.program_id(1) == 0)

    @pl.when(first)
    def _():
        stats_ref[...] = jnp.zeros_like(stats_ref)

    stats_ref[0:1, :] += jnp.sum(t0, axis=0, keepdims=True)
    stats_ref[1:2, :] += jnp.sum(t0 * t0, axis=0, keepdims=True)


def _bn_scale_shift(stats_ref, gb_ref, n_total, eps=1e-5):
    mean = stats_ref[0:1, :] * (1.0 / n_total)
    ex2 = stats_ref[1:2, :] * (1.0 / n_total)
    var = ex2 - mean * mean
    s = gb_ref[0:1, :] * jax.lax.rsqrt(var + eps)
    sh = gb_ref[1:2, :] - mean * s
    return s, sh


def _pass2_body(t0_ref, st_ref, gb_ref, w1t_ref, b1_ref, y1_ref, stats_ref,
                *, n_total):
    s, sh = _bn_scale_shift(st_ref, gb_ref, n_total)
    h = t0_ref[...] * s + sh
    h = jnp.maximum(h, 0.0)
    y1 = jnp.dot(h, w1t_ref[...],
                 preferred_element_type=jnp.float32) + b1_ref[0:1, :]
    y1_ref[...] = y1

    @pl.when(pl.program_id(0) == 0)
    def _():
        stats_ref[...] = jnp.zeros_like(stats_ref)

    stats_ref[0:1, :] += jnp.sum(y1, axis=0, keepdims=True)
    stats_ref[1:2, :] += jnp.sum(y1 * y1, axis=0, keepdims=True)


def _pass3_body(y1_ref, st_ref, gb_ref, out_ref, *, n_total):
    s, sh = _bn_scale_shift(st_ref, gb_ref, n_total)
    h = y1_ref[0] * s + sh
    h = jnp.maximum(h, 0.0)
    out_ref[0] = h.T


def kernel(xyz1, xyz2, points1, points2, W0, b0, g0, be0, W1, b1, g1, be1):
    b, n, _ = xyz1.shape
    m = xyz2.shape[1]
    c1 = points1.shape[2]
    c2 = points2.shape[2]
    cin = c1 + c2
    c_mid = W0.shape[0]
    c_out = W1.shape[0]
    N = b * n
    eps = 1e-5

    NB1 = 512
    NB2 = 512
    NB3 = 512
    nblk1 = n // NB1

    xyz2t = jnp.transpose(xyz2, (0, 2, 1))
    x2sq = jnp.sum(xyz2 * xyz2, axis=-1)[:, None, :]
    bmat = jnp.concatenate(
        [x2sq, jnp.ones((b, 1, m), jnp.float32), xyz2t,
         jnp.zeros((b, 3, m), jnp.float32)], axis=1)
    iota = jnp.broadcast_to(jnp.arange(m, dtype=jnp.int32)[None, :],
                            (NB1, m))
    w0t = W0.T
    w1t = W1.T
    b0r = jnp.broadcast_to(b0[None, :], (8, c_mid))
    b1r = jnp.broadcast_to(b1[None, :], (8, c_out))
    gb0 = jnp.concatenate([g0[None, :], be0[None, :],
                           jnp.zeros((6, c_mid), jnp.float32)], axis=0)
    gb1 = jnp.concatenate([g1[None, :], be1[None, :],
                           jnp.zeros((6, c_out), jnp.float32)], axis=0)

    t0, stats0 = pl.pallas_call(
        _pass1_body,
        grid=(b, nblk1),
        in_specs=[
            pl.BlockSpec((1, NB1, 3), lambda i, j: (i, j, 0)),
            pl.BlockSpec((1, 8, m), lambda i, j: (i, 0, 0)),
            pl.BlockSpec((NB1, m), lambda i, j: (0, 0)),
            pl.BlockSpec((1, m, c2), lambda i, j: (i, 0, 0)),
            pl.BlockSpec((1, NB1, c1), lambda i, j: (i, j, 0)),
            pl.BlockSpec((cin, c_mid), lambda i, j: (0, 0)),
            pl.BlockSpec((8, c_mid), lambda i, j: (0, 0)),
        ],
        out_specs=[
            pl.BlockSpec((NB1, c_mid), lambda i, j: (i * nblk1 + j, 0)),
            pl.BlockSpec((8, c_mid), lambda i, j: (0, 0)),
        ],
        out_shape=[
            jax.ShapeDtypeStruct((N, c_mid), jnp.float32),
            jax.ShapeDtypeStruct((8, c_mid), jnp.float32),
        ],
    )(xyz1, bmat, iota, points2, points1, w0t, b0r)

    y1, stats1 = pl.pallas_call(
        functools.partial(_pass2_body, n_total=N),
        grid=(N // NB2,),
        in_specs=[
            pl.BlockSpec((NB2, c_mid), lambda i: (i, 0)),
            pl.BlockSpec((8, c_mid), lambda i: (0, 0)),
            pl.BlockSpec((8, c_mid), lambda i: (0, 0)),
            pl.BlockSpec((c_mid, c_out), lambda i: (0, 0)),
            pl.BlockSpec((8, c_out), lambda i: (0, 0)),
        ],
        out_specs=[
            pl.BlockSpec((NB2, c_out), lambda i: (i, 0)),
            pl.BlockSpec((8, c_out), lambda i: (0, 0)),
        ],
        out_shape=[
            jax.ShapeDtypeStruct((N, c_out), jnp.float32),
            jax.ShapeDtypeStruct((8, c_out), jnp.float32),
        ],
    )(t0, stats0, gb0, w1t, b1r)

    out = pl.pallas_call(
        functools.partial(_pass3_body, n_total=N),
        grid=(b, n // NB3),
        in_specs=[
            pl.BlockSpec((1, NB3, c_out), lambda i, j: (i, j, 0)),
            pl.BlockSpec((8, c_out), lambda i, j: (0, 0)),
            pl.BlockSpec((8, c_out), lambda i, j: (0, 0)),
        ],
        out_specs=pl.BlockSpec((1, c_out, NB3), lambda i, j: (i, 0, j)),
        out_shape=jax.ShapeDtypeStruct((b, c_out, n), jnp.float32),
    )(y1.reshape(b, n, c_out), stats1, gb1)

    return out


# speedup vs baseline: 1.0644x; 1.0644x over previous
import functools

import jax
import jax.numpy as jnp
from jax.experimental import pallas as pl


def _pass1_body(a_ref, bmat_ref, iota_ref, p2_ref, p1_ref, w0t_ref, b0_ref,
                t0_ref, stats_ref):
    nb = a_ref.shape[1]
    m = bmat_ref.shape[2]
    c2 = p2_ref.shape[2]

    d2 = jnp.dot(a_ref[0], bmat_ref[0],
                 preferred_element_type=jnp.float32)

    key = jax.lax.bitcast_convert_type(
        (jax.lax.bitcast_convert_type(d2, jnp.int32)
         & jnp.int32(-2048)) | iota_ref[...], jnp.float32)
    inf = jnp.float32(jnp.inf)

    k1 = jnp.min(key, axis=1, keepdims=True)
    eq1 = key == k1
    key2 = jnp.where(eq1, inf, key)
    k2 = jnp.min(key2, axis=1, keepdims=True)
    eq2 = key2 == k2
    key3 = jnp.where(eq2, inf, key2)
    k3 = jnp.min(key3, axis=1, keepdims=True)
    eq3 = key3 == k3

    def weight(k):
        v = jax.lax.bitcast_convert_type(
            jax.lax.bitcast_convert_type(k, jnp.int32) & jnp.int32(-2048),
            jnp.float32)
        return 1.0 / jnp.clip(v, 0.0, 1e-10)

    r1, r2, r3 = weight(k1), weight(k2), weight(k3)
    inv_norm = 1.0 / (r1 + r2 + r3)
    w1, w2, w3 = r1 * inv_norm, r2 * inv_norm, r3 * inv_norm
    zero = jnp.zeros((nb, m), jnp.float32)
    wmat = jnp.where(eq1, w1,
                     jnp.where(eq2, w2, jnp.where(eq3, w3, zero)))
    interp = jnp.dot(wmat, p2_ref[0],
                     preferred_element_type=jnp.float32)
    t0 = (jnp.dot(interp, w0t_ref[:c2, :],
                  preferred_element_type=jnp.float32)
          + jnp.dot(p1_ref[0], w0t_ref[c2:, :],
                    preferred_element_type=jnp.float32)
          + b0_ref[0:1, :])
    t0_ref[...] = t0

    first = (pl.program_id(0) == 0) & (pl.program_id(1) == 0)

    @pl.when(first)
    def _():
        stats_ref[...] = jnp.zeros_like(stats_ref)

    stats_ref[0:1, :] += jnp.sum(t0, axis=0, keepdims=True)
    stats_ref[1:2, :] += jnp.sum(t0 * t0, axis=0, keepdims=True)


def _bn_scale_shift(stats_ref, gb_ref, n_total, eps=1e-5):
    mean = stats_ref[0:1, :] * (1.0 / n_total)
    ex2 = stats_ref[1:2, :] * (1.0 / n_total)
    var = ex2 - mean * mean
    s = gb_ref[0:1, :] * jax.lax.rsqrt(var + eps)
    sh = gb_ref[1:2, :] - mean * s
    return s, sh


def _pass2_body(t0_ref, st_ref, gb_ref, w1t_ref, b1_ref, y1_ref, stats_ref,
                *, n_total):
    s, sh = _bn_scale_shift(st_ref, gb_ref, n_total)
    h = t0_ref[...] * s + sh
    h = jnp.maximum(h, 0.0)
    y1 = jnp.dot(h, w1t_ref[...],
                 preferred_element_type=jnp.float32) + b1_ref[0:1, :]
    y1_ref[...] = y1

    @pl.when(pl.program_id(0) == 0)
    def _():
        stats_ref[...] = jnp.zeros_like(stats_ref)

    stats_ref[0:1, :] += jnp.sum(y1, axis=0, keepdims=True)
    stats_ref[1:2, :] += jnp.sum(y1 * y1, axis=0, keepdims=True)


def _pass3_body(y1_ref, st_ref, gb_ref, out_ref, *, n_total):
    s, sh = _bn_scale_shift(st_ref, gb_ref, n_total)
    h = y1_ref[0] * s + sh
    h = jnp.maximum(h, 0.0)
    out_ref[0] = h.T


def kernel(xyz1, xyz2, points1, points2, W0, b0, g0, be0, W1, b1, g1, be1):
    b, n, _ = xyz1.shape
    m = xyz2.shape[1]
    c1 = points1.shape[2]
    c2 = points2.shape[2]
    cin = c1 + c2
    c_mid = W0.shape[0]
    c_out = W1.shape[0]
    N = b * n
    eps = 1e-5

    NB1 = 512
    NB2 = 512
    NB3 = 512
    nblk1 = n // NB1

    x1sq = jnp.sum(xyz1 * xyz1, axis=-1, keepdims=True)
    amat = jnp.concatenate(
        [jnp.ones((b, n, 1), jnp.float32), x1sq, -2.0 * xyz1,
         jnp.zeros((b, n, 3), jnp.float32)], axis=-1)
    xyz2t = jnp.transpose(xyz2, (0, 2, 1))
    x2sq = jnp.sum(xyz2 * xyz2, axis=-1)[:, None, :]
    bmat = jnp.concatenate(
        [x2sq, jnp.ones((b, 1, m), jnp.float32), xyz2t,
         jnp.zeros((b, 3, m), jnp.float32)], axis=1)
    iota = jnp.broadcast_to(jnp.arange(m, dtype=jnp.int32)[None, :],
                            (NB1, m))
    w0t = W0.T
    w1t = W1.T
    b0r = jnp.broadcast_to(b0[None, :], (8, c_mid))
    b1r = jnp.broadcast_to(b1[None, :], (8, c_out))
    gb0 = jnp.concatenate([g0[None, :], be0[None, :],
                           jnp.zeros((6, c_mid), jnp.float32)], axis=0)
    gb1 = jnp.concatenate([g1[None, :], be1[None, :],
                           jnp.zeros((6, c_out), jnp.float32)], axis=0)

    t0, stats0 = pl.pallas_call(
        _pass1_body,
        grid=(b, nblk1),
        in_specs=[
            pl.BlockSpec((1, NB1, 8), lambda i, j: (i, j, 0)),
            pl.BlockSpec((1, 8, m), lambda i, j: (i, 0, 0)),
            pl.BlockSpec((NB1, m), lambda i, j: (0, 0)),
            pl.BlockSpec((1, m, c2), lambda i, j: (i, 0, 0)),
            pl.BlockSpec((1, NB1, c1), lambda i, j: (i, j, 0)),
            pl.BlockSpec((cin, c_mid), lambda i, j: (0, 0)),
            pl.BlockSpec((8, c_mid), lambda i, j: (0, 0)),
        ],
        out_specs=[
            pl.BlockSpec((NB1, c_mid), lambda i, j: (i * nblk1 + j, 0)),
            pl.BlockSpec((8, c_mid), lambda i, j: (0, 0)),
        ],
        out_shape=[
            jax.ShapeDtypeStruct((N, c_mid), jnp.float32),
            jax.ShapeDtypeStruct((8, c_mid), jnp.float32),
        ],
    )(amat, bmat, iota, points2, points1, w0t, b0r)

    y1, stats1 = pl.pallas_call(
        functools.partial(_pass2_body, n_total=N),
        grid=(N // NB2,),
        in_specs=[
            pl.BlockSpec((NB2, c_mid), lambda i: (i, 0)),
            pl.BlockSpec((8, c_mid), lambda i: (0, 0)),
            pl.BlockSpec((8, c_mid), lambda i: (0, 0)),
            pl.BlockSpec((c_mid, c_out), lambda i: (0, 0)),
            pl.BlockSpec((8, c_out), lambda i: (0, 0)),
        ],
        out_specs=[
            pl.BlockSpec((NB2, c_out), lambda i: (i, 0)),
            pl.BlockSpec((8, c_out), lambda i: (0, 0)),
        ],
        out_shape=[
            jax.ShapeDtypeStruct((N, c_out), jnp.float32),
            jax.ShapeDtypeStruct((8, c_out), jnp.float32),
        ],
    )(t0, stats0, gb0, w1t, b1r)

    out = pl.pallas_call(
        functools.partial(_pass3_body, n_total=N),
        grid=(b, n // NB3),
        in_specs=[
            pl.BlockSpec((1, NB3, c_out), lambda i, j: (i, j, 0)),
            pl.BlockSpec((8, c_out), lambda i, j: (0, 0)),
            pl.BlockSpec((8, c_out), lambda i, j: (0, 0)),
        ],
        out_specs=pl.BlockSpec((1, c_out, NB3), lambda i, j: (i, 0, j)),
        out_shape=jax.ShapeDtypeStruct((b, c_out, n), jnp.float32),
    )(y1.reshape(b, n, c_out), stats1, gb1)

    return out
